# baseline (speedup 1.0000x reference)
import functools

import jax
import jax.numpy as jnp
from jax.experimental import pallas as pl
from jax.experimental.pallas import tpu as pltpu

B, S, D, E, H, TOP_K = 1, 2048, 768, 8, 3072, 2

SBLK = 512
HBLK = 768
RBLK = 256
NEG_INF = float("-inf")


def _router_body(x_ref, wr_ref, br_ref, wn_ref, bn_ref, noise_ref, gate_ref):
    xb = x_ref[...]
    lg = jnp.dot(xb, wr_ref[...], preferred_element_type=jnp.float32) + br_ref[...]
    nl = jnp.dot(xb, wn_ref[...], preferred_element_type=jnp.float32) + bn_ref[...]
    sp = jnp.maximum(nl, 0.0) + jnp.log1p(jnp.exp(-jnp.abs(nl)))
    noisy = lg + noise_ref[...] * sp
    m1 = jnp.max(noisy, axis=1, keepdims=True)
    n2 = jnp.where(noisy == m1, NEG_INF, noisy)
    m2 = jnp.max(n2, axis=1, keepdims=True)
    sel = noisy >= m2
    denom = 1.0 + jnp.exp(m2 - m1)
    gate_ref[...] = jnp.where(sel, jnp.exp(noisy - m1), 0.0) / denom


def _ffn_body(x_ref, w1_ref, b1_ref, w2_ref, b2_ref, g_ref,
              exp_ref, fin_ref, acc_ref, *, nh):
    e_i = pl.program_id(1)
    h_i = pl.program_id(2)
    hblk = jax.nn.relu(
        jnp.dot(x_ref[...], w1_ref[0], preferred_element_type=jnp.float32)
        + b1_ref[0])
    part = jnp.dot(hblk, w2_ref[0], preferred_element_type=jnp.float32)

    @pl.when(h_i == 0)
    def _():
        acc_ref[...] = part

    @pl.when(h_i != 0)
    def _():
        acc_ref[...] = acc_ref[...] + part

    @pl.when(h_i == nh - 1)
    def _():
        eo = (acc_ref[...] + b2_ref[0]) * g_ref[0]
        exp_ref[0] = eo

        @pl.when(e_i == 0)
        def _():
            fin_ref[...] = eo

        @pl.when(e_i != 0)
        def _():
            fin_ref[...] = fin_ref[...] + eo


@jax.jit
def kernel(x, noise, Wr, br, Wn, bn, W1, b1, W2, b2):
    x2 = x.reshape(S, D)
    noise2 = noise.reshape(S, E)

    gate = pl.pallas_call(
        _router_body,
        grid=(S // RBLK,),
        in_specs=[
            pl.BlockSpec((RBLK, D), lambda i: (i, 0)),
            pl.BlockSpec((D, E), lambda i: (0, 0)),
            pl.BlockSpec((E,), lambda i: (0,)),
            pl.BlockSpec((D, E), lambda i: (0, 0)),
            pl.BlockSpec((E,), lambda i: (0,)),
            pl.BlockSpec((RBLK, E), lambda i: (i, 0)),
        ],
        out_specs=pl.BlockSpec((RBLK, E), lambda i: (i, 0)),
        out_shape=jax.ShapeDtypeStruct((S, E), jnp.float32),
    )(x2, Wr, br, Wn, bn, noise2)

    gcol = jnp.transpose(gate).reshape(E, S, 1)

    ns, nh = S // SBLK, H // HBLK
    exp_flat, fin = pl.pallas_call(
        functools.partial(_ffn_body, nh=nh),
        grid=(ns, E, nh),
        in_specs=[
            pl.BlockSpec((SBLK, D), lambda s, e, h: (s, 0)),
            pl.BlockSpec((1, D, HBLK), lambda s, e, h: (e, 0, h)),
            pl.BlockSpec((1, 1, HBLK), lambda s, e, h: (e, 0, h)),
            pl.BlockSpec((1, HBLK, D), lambda s, e, h: (e, h, 0)),
            pl.BlockSpec((1, 1, D), lambda s, e, h: (e, 0, 0)),
            pl.BlockSpec((1, SBLK, 1), lambda s, e, h: (e, s, 0)),
        ],
        out_specs=[
            pl.BlockSpec((1, SBLK, D), lambda s, e, h: (e, s, 0)),
            pl.BlockSpec((SBLK, D), lambda s, e, h: (s, 0)),
        ],
        out_shape=[
            jax.ShapeDtypeStruct((E, S, D), jnp.float32),
            jax.ShapeDtypeStruct((S, D), jnp.float32),
        ],
        scratch_shapes=[pltpu.VMEM((SBLK, D), jnp.float32)],
        compiler_params=pltpu.CompilerParams(
            dimension_semantics=("parallel", "arbitrary", "arbitrary"),
        ),
    )(x2, W1, b1.reshape(E, 1, H), W2, b2.reshape(E, 1, D), gcol)

    return (fin.reshape(B, S, D),
            exp_flat.reshape(E, B, S, D),
            gate.reshape(B, S, E))


# speedup vs baseline: 1.3812x; 1.3812x over previous
import functools

import jax
import jax.numpy as jnp
from jax.experimental import pallas as pl
from jax.experimental.pallas import tpu as pltpu

B, S, D, E, H, TOP_K = 1, 2048, 768, 8, 3072, 2

SBLK = 2048
HBLK = 768
RBLK = 256
NEG_INF = float("-inf")


def _router_body(x_ref, wr_ref, br_ref, wn_ref, bn_ref, noise_ref, gate_ref):
    xb = x_ref[...]
    lg = jnp.dot(xb, wr_ref[...], preferred_element_type=jnp.float32) + br_ref[...]
    nl = jnp.dot(xb, wn_ref[...], preferred_element_type=jnp.float32) + bn_ref[...]
    sp = jnp.maximum(nl, 0.0) + jnp.log1p(jnp.exp(-jnp.abs(nl)))
    noisy = lg + noise_ref[...] * sp
    m1 = jnp.max(noisy, axis=1, keepdims=True)
    n2 = jnp.where(noisy == m1, NEG_INF, noisy)
    m2 = jnp.max(n2, axis=1, keepdims=True)
    sel = noisy >= m2
    denom = 1.0 + jnp.exp(m2 - m1)
    gate_ref[...] = jnp.where(sel, jnp.exp(noisy - m1), 0.0) / denom


def _ffn_body(x_ref, w1_ref, b1_ref, w2_ref, b2_ref, g_ref,
              exp_ref, fin_ref, acc_ref, *, nh):
    e_i = pl.program_id(1)
    h_i = pl.program_id(2)
    hblk = jax.nn.relu(
        jnp.dot(x_ref[...], w1_ref[0], preferred_element_type=jnp.float32)
        + b1_ref[0])
    part = jnp.dot(hblk, w2_ref[0], preferred_element_type=jnp.float32)

    @pl.when(h_i == 0)
    def _():
        acc_ref[...] = part

    @pl.when(h_i != 0)
    def _():
        acc_ref[...] = acc_ref[...] + part

    @pl.when(h_i == nh - 1)
    def _():
        eo = (acc_ref[...] + b2_ref[0]) * g_ref[0]
        exp_ref[0] = eo

        @pl.when(e_i == 0)
        def _():
            fin_ref[...] = eo

        @pl.when(e_i != 0)
        def _():
            fin_ref[...] = fin_ref[...] + eo


@jax.jit
def kernel(x, noise, Wr, br, Wn, bn, W1, b1, W2, b2):
    x2 = x.reshape(S, D)
    noise2 = noise.reshape(S, E)

    gate = pl.pallas_call(
        _router_body,
        grid=(S // RBLK,),
        in_specs=[
            pl.BlockSpec((RBLK, D), lambda i: (i, 0)),
            pl.BlockSpec((D, E), lambda i: (0, 0)),
            pl.BlockSpec((E,), lambda i: (0,)),
            pl.BlockSpec((D, E), lambda i: (0, 0)),
            pl.BlockSpec((E,), lambda i: (0,)),
            pl.BlockSpec((RBLK, E), lambda i: (i, 0)),
        ],
        out_specs=pl.BlockSpec((RBLK, E), lambda i: (i, 0)),
        out_shape=jax.ShapeDtypeStruct((S, E), jnp.float32),
    )(x2, Wr, br, Wn, bn, noise2)

    gcol = jnp.transpose(gate).reshape(E, S, 1)

    ns, nh = S // SBLK, H // HBLK
    exp_flat, fin = pl.pallas_call(
        functools.partial(_ffn_body, nh=nh),
        grid=(ns, E, nh),
        in_specs=[
            pl.BlockSpec((SBLK, D), lambda s, e, h: (s, 0)),
            pl.BlockSpec((1, D, HBLK), lambda s, e, h: (e, 0, h)),
            pl.BlockSpec((1, 1, HBLK), lambda s, e, h: (e, 0, h)),
            pl.BlockSpec((1, HBLK, D), lambda s, e, h: (e, h, 0)),
            pl.BlockSpec((1, 1, D), lambda s, e, h: (e, 0, 0)),
            pl.BlockSpec((1, SBLK, 1), lambda s, e, h: (e, s, 0)),
        ],
        out_specs=[
            pl.BlockSpec((1, SBLK, D), lambda s, e, h: (e, s, 0)),
            pl.BlockSpec((SBLK, D), lambda s, e, h: (s, 0)),
        ],
        out_shape=[
            jax.ShapeDtypeStruct((E, S, D), jnp.float32),
            jax.ShapeDtypeStruct((S, D), jnp.float32),
        ],
        scratch_shapes=[pltpu.VMEM((SBLK, D), jnp.float32)],
        compiler_params=pltpu.CompilerParams(
            dimension_semantics=("parallel", "arbitrary", "arbitrary"),
        ),
    )(x2, W1, b1.reshape(E, 1, H), W2, b2.reshape(E, 1, D), gcol)

    return (fin.reshape(B, S, D),
            exp_flat.reshape(E, B, S, D),
            gate.reshape(B, S, E))
